# baseline (speedup 1.0000x reference)
import functools

import jax
import jax.numpy as jnp
from jax import lax
from jax.experimental import pallas as pl
from jax.experimental.pallas import tpu as pltpu
from jax.experimental.pallas import tpu_sc as plsc

K = 16
OB = 4
RBC = 8
NBUF = 4


def _make_sc_call(n_rows: int):
    info = plsc.get_sparse_core_info()
    nw = info.num_cores * info.num_subcores
    nb = n_rows // 128
    rb_w = nb // nw
    assert rb_w * nw == nb and rb_w % (RBC * NBUF) == 0
    nchunk = rb_w // RBC

    mesh = plsc.VectorSubcoreMesh(core_axis_name="c", subcore_axis_name="s")

    @functools.partial(
        pl.kernel,
        out_type=jax.ShapeDtypeStruct((nb, OB, 128), jnp.float32),
        mesh=mesh,
        scratch_types=(
            [pltpu.VMEM((2, RBC, 8, 128), jnp.float32)] * NBUF
            + [pltpu.VMEM((RBC, OB, 128), jnp.float32)] * NBUF
            + [pltpu.VMEM((K, OB), jnp.float32)]
            + [pltpu.SemaphoreType.DMA] * (2 * NBUF)
        ),
        compiler_params=pltpu.CompilerParams(
            needs_layout_passes=False, use_tc_tiling_on_sc=False),
    )
    def sc_kernel(x_hbm, b_hbm, out_hbm, *scratch):
        inbufs = scratch[:NBUF]
        outbufs = scratch[NBUF:2 * NBUF]
        bv = scratch[2 * NBUF]
        insems = scratch[2 * NBUF + 1:3 * NBUF + 1]
        outsems = scratch[3 * NBUF + 1:]

        wid = lax.axis_index("s") * info.num_cores + lax.axis_index("c")
        rb0 = wid * rb_w

        pltpu.sync_copy(b_hbm, bv)

        def copy_in(ci, b):
            start = rb0 + ci * RBC
            for half in range(2):
                pltpu.async_copy(
                    x_hbm.at[half, pl.ds(start, RBC)], inbufs[b].at[half],
                    insems[b])

        def copy_out(ci, b):
            pltpu.async_copy(
                outbufs[b], out_hbm.at[pl.ds(rb0 + ci * RBC, RBC)],
                outsems[b])

        def wait_in(b):
            for half in range(2):
                pltpu.make_async_copy(
                    x_hbm.at[half, pl.ds(0, RBC)], inbufs[b].at[half],
                    insems[b]).wait()

        def wait_out(b):
            pltpu.make_async_copy(
                outbufs[b], out_hbm.at[pl.ds(0, RBC)], outsems[b]).wait()

        col_consts = [jnp.full((K,), c, jnp.int32) for c in range(K)]

        def compute(in_ref, out_ref):
            @plsc.parallel_loop(0, RBC * 8, 1, unroll=2)
            def _grp(g):
                rbl = g >> 3
                sl = pl.ds((g & 7) * 16, 16)
                ms = [in_ref[c // 8, rbl, c % 8, sl] for c in range(K)]
                ixs = col_consts
                while len(ms) > 1:
                    nm, ni = [], []
                    for a in range(0, len(ms), 2):
                        pred = ms[a + 1] > ms[a]
                        nm.append(jnp.where(pred, ms[a + 1], ms[a]))
                        ni.append(jnp.where(pred, ixs[a + 1], ixs[a]))
                    ms, ixs = nm, ni
                idxv = ixs[0]
                for j in range(OB):
                    out_ref[rbl, j, sl] = plsc.load_gather(
                        bv, [idxv, col_consts[j]])

        for b in range(NBUF):
            copy_in(b, b)

        @pl.loop(0, nchunk // NBUF)
        def _ring(i):
            for b in range(NBUF):
                ci = i * NBUF + b
                wait_in(b)
                pl.when(ci >= NBUF)(lambda: wait_out(b))
                compute(inbufs[b], outbufs[b])
                copy_out(ci, b)
                pl.when(ci + NBUF < nchunk)(
                    lambda: copy_in(ci + NBUF, b))

        for b in range(NBUF):
            wait_out(b)

    return sc_kernel


@jax.jit
def kernel(decimal_tensor, B):
    n = decimal_tensor.shape[0]
    nb = n // 128
    x4 = decimal_tensor.reshape(nb, 128, 2, 8).transpose(2, 0, 3, 1)
    out4 = _make_sc_call(n)(x4, B)
    return out4.transpose(0, 2, 1).reshape(n, 1, OB)


# speedup vs baseline: 1.0280x; 1.0280x over previous
import functools

import jax
import jax.numpy as jnp
from jax import lax
from jax.experimental import pallas as pl
from jax.experimental.pallas import tpu as pltpu
from jax.experimental.pallas import tpu_sc as plsc

K = 16
OB = 4
RBC = 16


def _make_sc_call(n_rows: int):
    info = plsc.get_sparse_core_info()
    nw = info.num_cores * info.num_subcores
    nb = n_rows // 128
    rb_w = nb // nw
    assert rb_w * nw == nb and rb_w % RBC == 0
    nchunk = rb_w // RBC

    mesh = plsc.VectorSubcoreMesh(core_axis_name="c", subcore_axis_name="s")

    @functools.partial(
        pl.kernel,
        out_type=jax.ShapeDtypeStruct((nb, OB, 128), jnp.float32),
        mesh=mesh,
        scratch_types=[
            pltpu.VMEM((2, RBC, 8, 128), jnp.float32),
            pltpu.VMEM((2, RBC, 8, 128), jnp.float32),
            pltpu.VMEM((RBC, OB, 128), jnp.float32),
            pltpu.VMEM((RBC, OB, 128), jnp.float32),
            pltpu.VMEM((K, OB), jnp.float32),
            pltpu.SemaphoreType.DMA,
            pltpu.SemaphoreType.DMA,
            pltpu.SemaphoreType.DMA,
            pltpu.SemaphoreType.DMA,
        ],
        compiler_params=pltpu.CompilerParams(
            needs_layout_passes=False, use_tc_tiling_on_sc=False),
    )
    def sc_kernel(x_hbm, b_hbm, out_hbm, in0, in1, out0, out1, bv,
                  isem0, isem1, osem0, osem1):
        wid = lax.axis_index("s") * info.num_cores + lax.axis_index("c")
        rb0 = wid * rb_w

        inbufs, insems = (in0, in1), (isem0, isem1)
        outbufs, outsems = (out0, out1), (osem0, osem1)

        pltpu.sync_copy(b_hbm, bv)

        def copy_in(ci, buf, sem):
            start = rb0 + ci * RBC
            h0 = pltpu.async_copy(
                x_hbm.at[0, pl.ds(start, RBC)], buf.at[0], sem)
            h1 = pltpu.async_copy(
                x_hbm.at[1, pl.ds(start, RBC)], buf.at[1], sem)
            return (h0, h1)

        def copy_out(ci, buf, sem):
            return pltpu.async_copy(
                buf, out_hbm.at[pl.ds(rb0 + ci * RBC, RBC)], sem)

        col_consts = [jnp.full((K,), c, jnp.int32) for c in range(K)]

        def compute(in_ref, out_ref):
            @plsc.parallel_loop(0, RBC * 8, 1, unroll=2)
            def _grp(g):
                rbl = g >> 3
                sl = pl.ds((g & 7) * 16, 16)
                ms = [in_ref[c // 8, rbl, c % 8, sl] for c in range(K)]
                ixs = col_consts
                while len(ms) > 1:
                    nm, ni = [], []
                    for a in range(0, len(ms), 2):
                        pred = ms[a + 1] > ms[a]
                        nm.append(jnp.where(pred, ms[a + 1], ms[a]))
                        ni.append(jnp.where(pred, ixs[a + 1], ixs[a]))
                    ms, ixs = nm, ni
                idxv = ixs[0]
                for j in range(OB):
                    out_ref[rbl, j, sl] = plsc.load_gather(
                        bv, [idxv, col_consts[j]])

        def wait_in(b):
            for half in range(2):
                pltpu.make_async_copy(
                    x_hbm.at[half, pl.ds(0, RBC)], inbufs[b].at[half],
                    insems[b]).wait()

        def wait_out(b):
            pltpu.make_async_copy(
                outbufs[b], out_hbm.at[pl.ds(0, RBC)], outsems[b]).wait()

        assert nchunk % 2 == 0
        copy_in(0, in0, isem0)
        copy_in(1, in1, isem1)

        @pl.loop(0, nchunk // 2)
        def _ring(i):
            for b in range(2):
                ci = i * 2 + b
                wait_in(b)
                pl.when(ci >= 2)(lambda: wait_out(b))
                compute(inbufs[b], outbufs[b])
                copy_out(ci, outbufs[b], outsems[b])
                pl.when(ci + 2 < nchunk)(
                    lambda: (copy_in(ci + 2, inbufs[b], insems[b]), None)[1])

        for b in range(2):
            wait_out(b)

    return sc_kernel


@jax.jit
def kernel(decimal_tensor, B):
    n = decimal_tensor.shape[0]
    nb = n // 128
    x4 = decimal_tensor.reshape(nb, 128, 2, 8).transpose(2, 0, 3, 1)
    out4 = _make_sc_call(n)(x4, B)
    return out4.transpose(0, 2, 1).reshape(n, 1, OB)
